# Initial kernel scaffold; baseline (speedup 1.0000x reference)
import jax
import jax.numpy as jnp
from jax.experimental import pallas as pl


def kernel(x, bb0, bb1, bb2, bb3, w1, b1, proto):
    raise NotImplementedError("write your pallas kernel here")


# speedup vs baseline: 16.1970x; 16.1970x over previous
import jax
import jax.numpy as jnp
from jax.experimental import pallas as pl
from jax.experimental.pallas import tpu as pltpu

_HRNET_CHANNELS = (48, 96, 192, 384)
_PIXEL_MEAN = (123.675, 116.28, 103.53)
_PIXEL_STD = (58.395, 57.12, 57.375)

_HF = 128
_YB = 16
_NCLS = 19
_NCLS_PAD = 24


def _pyramid_kernel(x_ref, g_ref, gt_ref, u_ref):
    for c in range(3):
        xn = (x_ref[0, c] * (1.0 / _PIXEL_STD[c])
              + (-_PIXEL_MEAN[c] / _PIXEL_STD[c]))
        xn_bf = xn.astype(jnp.bfloat16)
        for k in range(4):
            a = jnp.dot(g_ref[k], xn_bf,
                        preferred_element_type=jnp.float32)
            ukc = jnp.dot(a, gt_ref[k],
                          preferred_element_type=jnp.float32)
            u_ref[0, 3 * k + c] = ukc.astype(jnp.bfloat16)
    u_ref[0, 12] = jnp.ones((_HF, _HF), jnp.bfloat16)
    u_ref[0, 13] = jnp.zeros((_HF, _HF), jnp.bfloat16)
    u_ref[0, 14] = jnp.zeros((_HF, _HF), jnp.bfloat16)
    u_ref[0, 15] = jnp.zeros((_HF, _HF), jnp.bfloat16)


def _head_kernel(u_ref, w1_ref, w2_ref, e_ref, out_ref):
    v = u_ref[0].reshape(16 * _YB, _HF)
    h = jnp.dot(w1_ref[...], v,
                preferred_element_type=jnp.float32)
    h = jnp.maximum(h, 0.0).astype(jnp.bfloat16)
    lg = jnp.dot(w2_ref[...], h,
                 preferred_element_type=jnp.float32)
    lge = jnp.dot(lg.astype(jnp.bfloat16), e_ref[...],
                  preferred_element_type=jnp.float32)
    t = lge[:_NCLS * _YB].reshape(_NCLS, _YB, 1, 4 * _HF)
    out_ref[0] = jnp.broadcast_to(
        t, (_NCLS, _YB, 4, 4 * _HF)).reshape(_NCLS, 4 * _YB, 4 * _HF)


def _resize_mat(n):
    return jax.image.resize(jnp.eye(n, dtype=jnp.float32), (_HF, n),
                            method='bilinear')


def _pool_mat(n):
    s = 512 // n
    return jnp.kron(jnp.eye(n, dtype=jnp.float32),
                    jnp.full((1, s), 1.0 / s, jnp.float32))


def kernel(x, bb0, bb1, bb2, bb3, w1, b1, proto):
    B, _, H, W = x.shape
    bb = [bb0, bb1, bb2, bb3]

    offs, rows = 0, []
    for k, ck in enumerate(_HRNET_CHANNELS):
        rows.append(bb[k] @ w1[offs:offs + ck])
        offs += ck
    wfold = jnp.concatenate(rows + [b1.reshape(1, -1).astype(jnp.float32),
                                    jnp.zeros((3, w1.shape[1]), jnp.float32)],
                            axis=0)
    wp = wfold.T
    pr = jnp.pad(proto[:, :_NCLS].T,
                 ((0, _NCLS_PAD - _NCLS), (0, 0)))
    eye_y = jnp.eye(_YB, dtype=jnp.float32)
    w1b = jnp.einsum('jc,yz->yjcz', wp, eye_y).reshape(
        _YB * 256, 16 * _YB).astype(jnp.bfloat16)
    w2b = jnp.einsum('cj,yz->cyzj', pr, eye_y).reshape(
        _NCLS_PAD * _YB, _YB * 256).astype(jnp.bfloat16)

    g = jnp.stack([_pool_mat(128),
                   _resize_mat(64) @ _pool_mat(64),
                   _resize_mat(32) @ _pool_mat(32),
                   _resize_mat(16) @ _pool_mat(16)])
    gt = jnp.swapaxes(g, 1, 2)
    g = g.astype(jnp.bfloat16)

    e = (jnp.arange(4 * _HF)[None, :] // 4
         == jnp.arange(_HF)[:, None]).astype(jnp.bfloat16)

    u = pl.pallas_call(
        _pyramid_kernel,
        out_shape=jax.ShapeDtypeStruct((B, 16, _HF, _HF), jnp.bfloat16),
        grid=(B,),
        in_specs=[pl.BlockSpec((1, 3, H, W), lambda b: (b, 0, 0, 0)),
                  pl.BlockSpec((4, _HF, W), lambda b: (0, 0, 0)),
                  pl.BlockSpec((4, W, _HF), lambda b: (0, 0, 0))],
        out_specs=pl.BlockSpec((1, 16, _HF, _HF), lambda b: (b, 0, 0, 0)),
        compiler_params=pltpu.CompilerParams(
            dimension_semantics=("parallel",)),
    )(x, g, gt)

    n_yb = _HF // _YB
    out = pl.pallas_call(
        _head_kernel,
        out_shape=jax.ShapeDtypeStruct((B, _NCLS, H, W), jnp.float32),
        grid=(B, n_yb),
        in_specs=[pl.BlockSpec((1, 16, _YB, _HF), lambda b, s: (b, 0, s, 0)),
                  pl.BlockSpec((_YB * 256, 16 * _YB), lambda b, s: (0, 0)),
                  pl.BlockSpec((_NCLS_PAD * _YB, _YB * 256),
                               lambda b, s: (0, 0)),
                  pl.BlockSpec((_HF, 4 * _HF), lambda b, s: (0, 0))],
        out_specs=pl.BlockSpec((1, _NCLS, 4 * _YB, W),
                               lambda b, s: (b, 0, s, 0)),
        compiler_params=pltpu.CompilerParams(
            dimension_semantics=("parallel", "parallel")),
    )(u, w1b, w2b, e)
    return out


# speedup vs baseline: 16.5930x; 1.0244x over previous
import numpy as np

import jax
import jax.numpy as jnp
from jax.experimental import pallas as pl
from jax.experimental.pallas import tpu as pltpu

_HRNET_CHANNELS = (48, 96, 192, 384)
_PIXEL_MEAN = (123.675, 116.28, 103.53)
_PIXEL_STD = (58.395, 57.12, 57.375)

_HF = 128
_YB = 16
_NCLS = 19
_NCLS_PAD = 24


def _pyramid_kernel(x_ref, g_ref, gt_ref, u_ref):
    for c in range(3):
        xn = (x_ref[0, c] * (1.0 / _PIXEL_STD[c])
              + (-_PIXEL_MEAN[c] / _PIXEL_STD[c]))
        xn_bf = xn.astype(jnp.bfloat16)
        for k in range(4):
            a = jnp.dot(g_ref[k], xn_bf,
                        preferred_element_type=jnp.float32)
            ukc = jnp.dot(a, gt_ref[k],
                          preferred_element_type=jnp.float32)
            u_ref[0, 3 * k + c] = ukc.astype(jnp.bfloat16)
    u_ref[0, 12] = jnp.ones((_HF, _HF), jnp.bfloat16)
    u_ref[0, 13] = jnp.zeros((_HF, _HF), jnp.bfloat16)
    u_ref[0, 14] = jnp.zeros((_HF, _HF), jnp.bfloat16)
    u_ref[0, 15] = jnp.zeros((_HF, _HF), jnp.bfloat16)


def _head_kernel(u_ref, w1_ref, w2_ref, e_ref, out_ref):
    v = u_ref[0].reshape(16 * _YB, _HF)
    h = jnp.dot(w1_ref[...], v,
                preferred_element_type=jnp.float32)
    h = jnp.maximum(h, 0.0).astype(jnp.bfloat16)
    lg = jnp.dot(w2_ref[...], h,
                 preferred_element_type=jnp.float32)
    lge = jnp.dot(lg.astype(jnp.bfloat16), e_ref[...],
                  preferred_element_type=jnp.float32)
    t = lge[:_NCLS * _YB].reshape(_NCLS, _YB, 1, 4 * _HF)
    out_ref[0] = jnp.broadcast_to(
        t, (_NCLS, _YB, 4, 4 * _HF)).reshape(_NCLS, 4 * _YB, 4 * _HF)


def _resize_mat(n):
    c = (np.arange(_HF) + 0.5) * n / _HF - 0.5
    w = np.maximum(0.0, 1.0 - np.abs(c[:, None] - np.arange(n)[None, :]))
    return (w / w.sum(axis=1, keepdims=True)).astype(np.float32)


def _pool_mat(n):
    s = 512 // n
    return np.kron(np.eye(n, dtype=np.float32),
                   np.full((1, s), 1.0 / s, np.float32))


def kernel(x, bb0, bb1, bb2, bb3, w1, b1, proto):
    B, _, H, W = x.shape
    bb = [bb0, bb1, bb2, bb3]

    offs, rows = 0, []
    for k, ck in enumerate(_HRNET_CHANNELS):
        rows.append(bb[k] @ w1[offs:offs + ck])
        offs += ck
    wfold = jnp.concatenate(rows + [b1.reshape(1, -1).astype(jnp.float32),
                                    jnp.zeros((3, w1.shape[1]), jnp.float32)],
                            axis=0)
    wp = wfold.T
    pr = jnp.pad(proto[:, :_NCLS].T,
                 ((0, _NCLS_PAD - _NCLS), (0, 0)))
    eye_y = jnp.eye(_YB, dtype=jnp.float32)
    w1b = jnp.einsum('jc,yz->yjcz', wp, eye_y).reshape(
        _YB * 256, 16 * _YB).astype(jnp.bfloat16)
    w2b = jnp.einsum('cj,yz->cyzj', pr, eye_y).reshape(
        _NCLS_PAD * _YB, _YB * 256).astype(jnp.bfloat16)

    g_np = np.stack([_pool_mat(128),
                     _resize_mat(64) @ _pool_mat(64),
                     _resize_mat(32) @ _pool_mat(32),
                     _resize_mat(16) @ _pool_mat(16)])
    gt = jnp.asarray(np.swapaxes(g_np, 1, 2))
    g = jnp.asarray(g_np.astype(jnp.bfloat16))

    e = jnp.asarray((np.arange(4 * _HF)[None, :] // 4
                     == np.arange(_HF)[:, None]).astype(jnp.bfloat16))

    u = pl.pallas_call(
        _pyramid_kernel,
        out_shape=jax.ShapeDtypeStruct((B, 16, _HF, _HF), jnp.bfloat16),
        grid=(B,),
        in_specs=[pl.BlockSpec((1, 3, H, W), lambda b: (b, 0, 0, 0)),
                  pl.BlockSpec((4, _HF, W), lambda b: (0, 0, 0)),
                  pl.BlockSpec((4, W, _HF), lambda b: (0, 0, 0))],
        out_specs=pl.BlockSpec((1, 16, _HF, _HF), lambda b: (b, 0, 0, 0)),
        compiler_params=pltpu.CompilerParams(
            dimension_semantics=("parallel",)),
    )(x, g, gt)

    n_yb = _HF // _YB
    out = pl.pallas_call(
        _head_kernel,
        out_shape=jax.ShapeDtypeStruct((B, _NCLS, H, W), jnp.float32),
        grid=(B, n_yb),
        in_specs=[pl.BlockSpec((1, 16, _YB, _HF), lambda b, s: (b, 0, s, 0)),
                  pl.BlockSpec((_YB * 256, 16 * _YB), lambda b, s: (0, 0)),
                  pl.BlockSpec((_NCLS_PAD * _YB, _YB * 256),
                               lambda b, s: (0, 0)),
                  pl.BlockSpec((_HF, 4 * _HF), lambda b, s: (0, 0))],
        out_specs=pl.BlockSpec((1, _NCLS, 4 * _YB, W),
                               lambda b, s: (b, 0, s, 0)),
        compiler_params=pltpu.CompilerParams(
            dimension_semantics=("parallel", "parallel")),
    )(u, w1b, w2b, e)
    return out
